# baseline (speedup 1.0000x reference)
import functools

import jax
import jax.numpy as jnp
from jax import lax
from jax.experimental import pallas as pl
from jax.experimental.pallas import tpu as pltpu
from jax.experimental.pallas import tpu_sc as plsc

_B, _H, _W = 8, 256, 256
_N_CLASSES = 313
_NB0, _NB1 = 6, 2


_TILE_H = 64
_GRID_H = _H // _TILE_H


def _argmax_body(cc_ref, idx_ref):
    m = cc_ref[0, 0]
    idx = jnp.zeros(m.shape, jnp.int32)
    for c in range(1, _N_CLASSES):
        v = cc_ref[0, c]
        cmp = v > m
        m = jnp.where(cmp, v, m)
        idx = jnp.where(cmp, jnp.int32(c), idx)
    idx_ref[...] = idx.reshape(1, _TILE_H, _W)


def _tc_argmax_half(cc_t, b_off, nb):
    return pl.pallas_call(
        _argmax_body,
        grid=(nb, _GRID_H),
        in_specs=[
            pl.BlockSpec(
                (1, _N_CLASSES, _TILE_H, _W),
                lambda b, h: (b + b_off, 0, h, 0),
            )
        ],
        out_specs=pl.BlockSpec((1, _TILE_H, _W), lambda b, h: (b, h, 0)),
        out_shape=jax.ShapeDtypeStruct((nb, _H, _W), jnp.int32),
    )(cc_t)


_NC = 2
_NS = 16
_NW = _NC * _NS
_CMAP_PAD = 640


def _build_sc_gather(nb):
    trow_per_w = nb * 2 * (_H // 8) // _NW
    mesh = plsc.VectorSubcoreMesh(core_axis_name="c", subcore_axis_name="s")

    @functools.partial(
        pl.kernel,
        mesh=mesh,
        out_type=jax.ShapeDtypeStruct((nb * _H * _W * 2,), jnp.float32),
        scratch_types=[
            pltpu.VMEM((2048,), jnp.int32),
            pltpu.VMEM((2048,), jnp.float32),
            pltpu.VMEM((_CMAP_PAD,), jnp.float32),
        ],
        compiler_params=pltpu.CompilerParams(
            use_tc_tiling_on_sc=False, needs_layout_passes=False
        ),
    )
    def sc_gather(idx_hbm, cmap_hbm, out_hbm, idx_v, out_v, cmap_v):
        wid = lax.axis_index("s") * _NC + lax.axis_index("c")
        pltpu.sync_copy(cmap_hbm, cmap_v)

        def tile_row(t, carry):
            big_t = wid * trow_per_w + t
            b = big_t >> 6
            c = (big_t >> 5) & 1
            hh = big_t & 31
            pltpu.sync_copy(idx_hbm.at[pl.ds((b * 32 + hh) * 2048, 2048)], idx_v)

            def body(n, carry2):
                idx16 = idx_v[pl.ds(n * 16, 16)]
                val = plsc.load_gather(cmap_v, [idx16 * 2 + c])
                out_v[pl.ds(n * 16, 16)] = val
                return carry2

            lax.fori_loop(0, 128, body, 0)
            pltpu.sync_copy(out_v, out_hbm.at[pl.ds(big_t * 2048, 2048)])
            return carry

        lax.fori_loop(0, trow_per_w, tile_row, 0)

    return sc_gather


_sc_gather_0 = _build_sc_gather(_NB0)
_sc_gather_1 = _build_sc_gather(_NB1)


def _idx_lin(idx, nb):
    return (
        idx.reshape(nb, _H // 8, 8, 2, 128)
        .transpose(0, 1, 3, 2, 4)
        .reshape(-1)
    )


def _ab_view(flat, nb):
    ab6 = flat.reshape(nb, 2, _H // 8, 2, 8, 128)
    return ab6.transpose(0, 1, 2, 4, 3, 5).reshape(nb, 2, _H, _W).transpose(
        0, 2, 3, 1
    )


def kernel(grayscale, color_classes, color_map):
    cc_t = color_classes.transpose(0, 3, 1, 2)
    cmap_flat = jnp.pad(color_map.reshape(-1), (0, _CMAP_PAD - 2 * _N_CLASSES))
    idx0 = _idx_lin(_tc_argmax_half(cc_t, 0, _NB0), _NB0)
    ab0 = _sc_gather_0(idx0, cmap_flat)
    idx1 = _idx_lin(_tc_argmax_half(cc_t, _NB0, _NB1), _NB1)
    ab1 = _sc_gather_1(idx1, cmap_flat)
    ab = jnp.concatenate([_ab_view(ab0, _NB0), _ab_view(ab1, _NB1)], axis=0)
    return jnp.concatenate([grayscale, ab], axis=-1)


# speedup vs baseline: 1.0208x; 1.0208x over previous
import functools

import jax
import jax.numpy as jnp
from jax import lax
from jax.experimental import pallas as pl
from jax.experimental.pallas import tpu as pltpu
from jax.experimental.pallas import tpu_sc as plsc

_B, _H, _W = 8, 256, 256
_N_PIX = _B * _H * _W
_N_CLASSES = 313
_BH = _B // 2


_TILE_H = 64
_GRID_H = _H // _TILE_H


def _argmax_body(cc_ref, idx_ref):
    m = cc_ref[0, 0]
    idx = jnp.zeros(m.shape, jnp.int32)
    for c in range(1, _N_CLASSES):
        v = cc_ref[0, c]
        cmp = v > m
        m = jnp.where(cmp, v, m)
        idx = jnp.where(cmp, jnp.int32(c), idx)
    idx_ref[...] = idx.reshape(1, _TILE_H, _W)


def _tc_argmax_half(cc_t, b_off):
    return pl.pallas_call(
        _argmax_body,
        grid=(_BH, _GRID_H),
        in_specs=[
            pl.BlockSpec(
                (1, _N_CLASSES, _TILE_H, _W),
                lambda b, h: (b + b_off, 0, h, 0),
            )
        ],
        out_specs=pl.BlockSpec((1, _TILE_H, _W), lambda b, h: (b, h, 0)),
        out_shape=jax.ShapeDtypeStruct((_BH, _H, _W), jnp.int32),
    )(cc_t)


_NC = 2
_NS = 16
_NW = _NC * _NS
_CMAP_PAD = 640
_N_TROW = _BH * 2 * (_H // 8)
_TROW_PER_W = _N_TROW // _NW


def _build_sc_gather():
    mesh = plsc.VectorSubcoreMesh(core_axis_name="c", subcore_axis_name="s")

    @functools.partial(
        pl.kernel,
        mesh=mesh,
        out_type=jax.ShapeDtypeStruct((_BH * _H * _W * 2,), jnp.float32),
        scratch_types=[
            pltpu.VMEM((2048,), jnp.int32),
            pltpu.VMEM((2048,), jnp.float32),
            pltpu.VMEM((_CMAP_PAD,), jnp.float32),
        ],
        compiler_params=pltpu.CompilerParams(
            use_tc_tiling_on_sc=False, needs_layout_passes=False
        ),
    )
    def sc_gather(idx_hbm, cmap_hbm, out_hbm, idx_v, out_v, cmap_v):
        wid = lax.axis_index("s") * _NC + lax.axis_index("c")
        pltpu.sync_copy(cmap_hbm, cmap_v)

        def tile_row(t, carry):
            big_t = wid * _TROW_PER_W + t
            b = big_t >> 6
            c = (big_t >> 5) & 1
            hh = big_t & 31
            pltpu.sync_copy(idx_hbm.at[pl.ds((b * 32 + hh) * 2048, 2048)], idx_v)

            def body(n, carry2):
                idx16 = idx_v[pl.ds(n * 16, 16)]
                val = plsc.load_gather(cmap_v, [idx16 * 2 + c])
                out_v[pl.ds(n * 16, 16)] = val
                return carry2

            lax.fori_loop(0, 128, body, 0)
            pltpu.sync_copy(out_v, out_hbm.at[pl.ds(big_t * 2048, 2048)])
            return carry

        lax.fori_loop(0, _TROW_PER_W, tile_row, 0)

    return sc_gather


_sc_gather = _build_sc_gather()


def _idx_lin(idx):
    return (
        idx.reshape(_BH, _H // 8, 8, 2, 128)
        .transpose(0, 1, 3, 2, 4)
        .reshape(-1)
    )


def _ab_view(flat):
    ab6 = flat.reshape(_BH, 2, _H // 8, 2, 8, 128)
    return ab6.transpose(0, 1, 2, 4, 3, 5).reshape(_BH, 2, _H, _W).transpose(
        0, 2, 3, 1
    )


def kernel(grayscale, color_classes, color_map):
    cc_t = color_classes.transpose(0, 3, 1, 2)
    cmap_flat = jnp.pad(color_map.reshape(-1), (0, _CMAP_PAD - 2 * _N_CLASSES))
    idx0 = _idx_lin(_tc_argmax_half(cc_t, 0))
    ab0 = _sc_gather(idx0, cmap_flat)
    idx1 = _idx_lin(_tc_argmax_half(cc_t, _BH))
    ab1 = _sc_gather(idx1, cmap_flat)
    ab = jnp.concatenate([_ab_view(ab0), _ab_view(ab1)], axis=0)
    return jnp.concatenate([grayscale, ab], axis=-1)
